# Initial kernel scaffold; baseline (speedup 1.0000x reference)
import jax
import jax.numpy as jnp
from jax.experimental import pallas as pl


def kernel(sub_index, derived_sub_indices, action_mask, table):
    raise NotImplementedError("write your pallas kernel here")


# speedup vs baseline: 3.9060x; 3.9060x over previous
import functools

import jax
import jax.numpy as jnp
from jax import lax
from jax.experimental import pallas as pl
from jax.experimental.pallas import tpu as pltpu
from jax.experimental.pallas import tpu_sc as plsc

B = 1024
S = 20
E = 128
ROWS_PER_SEG = 12
NUM_SEG = B * (S + 1)
NC, NS = 2, 16
NW = NC * NS
SEG_PER_W = NUM_SEG // NW
CHUNK_SEG = 8
CHUNK_ROWS = CHUNK_SEG * ROWS_PER_SEG
NCHUNK = SEG_PER_W // CHUNK_SEG
NGROUP = E // 16

_mesh = plsc.VectorSubcoreMesh(core_axis_name="c", subcore_axis_name="s")


@functools.partial(
    pl.kernel,
    out_type=jax.ShapeDtypeStruct((NUM_SEG, E), jnp.float32),
    mesh=_mesh,
    scratch_types=[
        pltpu.VMEM((NCHUNK, CHUNK_ROWS), jnp.int32),
        pltpu.VMEM((CHUNK_ROWS, E), jnp.float32),
        pltpu.VMEM((CHUNK_SEG, E), jnp.float32),
        pltpu.SemaphoreType.DMA,
    ],
)
def _embed_kernel(idx_hbm, table_hbm, out_hbm, idx_v, rows_v, out_v, gsem):
    wid = lax.axis_index("s") * NC + lax.axis_index("c")
    seg_base = wid * SEG_PER_W
    pltpu.sync_copy(idx_hbm.at[wid], idx_v)

    def chunk_body(j, _):
        pltpu.async_copy(table_hbm.at[idx_v.at[j]], rows_v, gsem).wait()

        def seg_body(s, _):
            rbase = s * ROWS_PER_SEG
            for g in range(NGROUP):
                sl = pl.ds(g * 16, 16)
                acc = rows_v[rbase, sl]
                for r in range(1, ROWS_PER_SEG):
                    acc = acc + rows_v[rbase + r, sl]
                out_v[s, sl] = acc * (1.0 / ROWS_PER_SEG)
            return 0

        lax.fori_loop(0, CHUNK_SEG, seg_body, 0)
        pltpu.sync_copy(out_v,
                        out_hbm.at[pl.ds(seg_base + j * CHUNK_SEG, CHUNK_SEG)])
        return 0

    lax.fori_loop(0, NCHUNK, chunk_body, 0)


def kernel(sub_index, derived_sub_indices, action_mask, table):
    idx_all = jnp.concatenate(
        [sub_index.astype(jnp.int32).reshape(-1),
         derived_sub_indices.astype(jnp.int32).reshape(-1)])
    idx_all = idx_all.reshape(NW, NCHUNK, CHUNK_ROWS)
    out = _embed_kernel(idx_all, table)
    obs = out[:B].reshape(B, 1, E)
    act = out[B:].reshape(B, S, E)
    return (obs, act, action_mask)


# speedup vs baseline: 5.4690x; 1.4002x over previous
import functools

import jax
import jax.numpy as jnp
from jax import lax
from jax.experimental import pallas as pl
from jax.experimental.pallas import tpu as pltpu
from jax.experimental.pallas import tpu_sc as plsc

B = 1024
S = 20
E = 128
ROWS_PER_SEG = 12
NUM_SEG = B * (S + 1)
NC, NS = 2, 16
NW = NC * NS
SEG_PER_W = NUM_SEG // NW
CHUNK_SEG = 8
CHUNK_ROWS = CHUNK_SEG * ROWS_PER_SEG
NCHUNK = SEG_PER_W // CHUNK_SEG
NGROUP = E // 16

_mesh = plsc.VectorSubcoreMesh(core_axis_name="c", subcore_axis_name="s")


@functools.partial(
    pl.kernel,
    out_type=jax.ShapeDtypeStruct((NUM_SEG, E), jnp.float32),
    mesh=_mesh,
    scratch_types=[
        pltpu.VMEM((NCHUNK, CHUNK_ROWS), jnp.int32),
        pltpu.VMEM((CHUNK_ROWS, E), jnp.float32),
        pltpu.VMEM((CHUNK_ROWS, E), jnp.float32),
        pltpu.VMEM((SEG_PER_W, E), jnp.float32),
        pltpu.SemaphoreType.DMA,
        pltpu.SemaphoreType.DMA,
    ],
)
def _embed_kernel(idx_hbm, table_hbm, out_hbm, idx_v, rows0, rows1, out_v,
                  gsem0, gsem1):
    wid = lax.axis_index("s") * NC + lax.axis_index("c")
    rows = (rows0, rows1)
    gsem = (gsem0, gsem1)
    pltpu.sync_copy(idx_hbm.at[wid], idx_v)
    pltpu.async_copy(table_hbm.at[idx_v.at[0]], rows0, gsem0)

    def pair_body(k, _):
        for b in range(2):
            j = 2 * k + b
            pltpu.make_async_copy(table_hbm.at[idx_v.at[j]], rows[b],
                                  gsem[b]).wait()
            nxt = j + 1

            @pl.when(nxt < NCHUNK)
            def _start_next():
                pltpu.async_copy(table_hbm.at[idx_v.at[nxt]], rows[1 - b],
                                 gsem[1 - b])

            def seg_body(s, _, b=b, j=j):
                rbase = s * ROWS_PER_SEG
                obase = j * CHUNK_SEG + s
                for g in range(NGROUP):
                    sl = pl.ds(g * 16, 16)
                    acc = rows[b][rbase, sl]
                    for r in range(1, ROWS_PER_SEG):
                        acc = acc + rows[b][rbase + r, sl]
                    out_v[obase, sl] = acc * (1.0 / ROWS_PER_SEG)
                return 0

            lax.fori_loop(0, CHUNK_SEG, seg_body, 0)
        return 0

    lax.fori_loop(0, NCHUNK // 2, pair_body, 0)
    pltpu.sync_copy(out_v, out_hbm.at[pl.ds(wid * SEG_PER_W, SEG_PER_W)])


def kernel(sub_index, derived_sub_indices, action_mask, table):
    idx_all = jnp.concatenate(
        [sub_index.astype(jnp.int32).reshape(-1),
         derived_sub_indices.astype(jnp.int32).reshape(-1)])
    idx_all = idx_all.reshape(NW, NCHUNK, CHUNK_ROWS)
    out = _embed_kernel(idx_all, table)
    obs = out[:B].reshape(B, 1, E)
    act = out[B:].reshape(B, S, E)
    return (obs, act, action_mask)
